# baseline (speedup 1.0000x reference)
import functools

import jax
import jax.numpy as jnp
from jax import lax
from jax.experimental import pallas as pl
from jax.experimental.pallas import tpu as pltpu
from jax.experimental.pallas import tpu_sc as plsc

_NC, _NS, _L = 1, 16, 16
_NW = _NC * _NS


_BM = 2048


def _argmin_body(x_ref, t_ref, ind_ref):
    t = t_ref[...]
    xt = x_ref[...].T
    dots = jnp.dot(t, xt, preferred_element_type=jnp.float32,
                   precision=lax.Precision.HIGHEST)
    tnorm = jnp.sum(t * t, axis=1)
    s = tnorm[:, None] - 2.0 * dots
    bmin = jnp.min(s, axis=0)
    iota = lax.broadcasted_iota(jnp.int32, s.shape, 0)
    masked = jnp.where(s == bmin[None, :], iota, jnp.int32(s.shape[0]))
    ind_ref[...] = jnp.min(masked, axis=0)


def _nearest_index(x, train_x):
    q = x.shape[0]
    m, dim = train_x.shape
    return pl.pallas_call(
        _argmin_body,
        out_shape=jax.ShapeDtypeStruct((q,), jnp.int32),
    )(x, train_x)


def _label_gather(ind, train_y):
    b = ind.shape[0]
    mesh = plsc.VectorSubcoreMesh(core_axis_name="c", subcore_axis_name="s", num_cores=1)

    @functools.partial(
        pl.kernel,
        mesh=mesh,
        out_type=jax.ShapeDtypeStruct((b,), jnp.float32),
        scratch_types=[
            pltpu.VMEM((b,), jnp.int32),
            pltpu.VMEM((b,), jnp.float32),
            pltpu.SemaphoreType.DMA,
        ],
    )
    def gather_kernel(ind_hbm, ty_hbm, out_hbm, idx_v, out_v, sem):
        wid = lax.axis_index("s") * _NC + lax.axis_index("c")

        @pl.when(wid == 0)
        def _():
            pltpu.sync_copy(ind_hbm, idx_v)
            pltpu.async_copy(ty_hbm.at[idx_v], out_v, sem).wait()
            pltpu.sync_copy(out_v, out_hbm)

    return gather_kernel(ind, train_y)


def kernel(x, train_x, train_y):
    ind = _nearest_index(x, train_x)
    return _label_gather(ind, train_y)


# speedup vs baseline: 1.0282x; 1.0282x over previous
import functools

import jax
import jax.numpy as jnp
from jax import lax
from jax.experimental import pallas as pl
from jax.experimental.pallas import tpu as pltpu
from jax.experimental.pallas import tpu_sc as plsc

_NC, _NS, _L = 1, 16, 16
_NW = _NC * _NS
_ACTIVE = 4


_BM = 2048


def _argmin_body(x_ref, t_ref, ind_ref):
    t = t_ref[...]
    xt = x_ref[...].T
    dots = jnp.dot(t, xt, preferred_element_type=jnp.float32,
                   precision=lax.Precision.HIGHEST)
    tnorm = jnp.sum(t * t, axis=1)
    s = tnorm[:, None] - 2.0 * dots
    bmin = jnp.min(s, axis=0)
    iota = lax.broadcasted_iota(jnp.int32, s.shape, 0)
    masked = jnp.where(s == bmin[None, :], iota, jnp.int32(s.shape[0]))
    ind_ref[...] = jnp.min(masked, axis=0)


def _nearest_index(x, train_x):
    q = x.shape[0]
    m, dim = train_x.shape
    return pl.pallas_call(
        _argmin_body,
        out_shape=jax.ShapeDtypeStruct((q,), jnp.int32),
    )(x, train_x)


def _label_gather(ind, train_y):
    b = ind.shape[0]
    b_per_w = b // _ACTIVE
    mesh = plsc.VectorSubcoreMesh(core_axis_name="c", subcore_axis_name="s", num_cores=1)

    @functools.partial(
        pl.kernel,
        mesh=mesh,
        out_type=jax.ShapeDtypeStruct((b,), jnp.float32),
        scratch_types=[
            pltpu.VMEM((b_per_w,), jnp.int32),
            pltpu.VMEM((b_per_w,), jnp.float32),
            pltpu.SemaphoreType.DMA,
        ],
    )
    def gather_kernel(ind_hbm, ty_hbm, out_hbm, idx_v, out_v, sem):
        wid = lax.axis_index("s") * _NC + lax.axis_index("c")
        base = wid * b_per_w

        @pl.when(wid < _ACTIVE)
        def _():
            pltpu.sync_copy(ind_hbm.at[pl.ds(base, b_per_w)], idx_v)
            pltpu.async_copy(ty_hbm.at[idx_v], out_v, sem).wait()
            pltpu.sync_copy(out_v, out_hbm.at[pl.ds(base, b_per_w)])

    return gather_kernel(ind, train_y)


def kernel(x, train_x, train_y):
    ind = _nearest_index(x, train_x)
    return _label_gather(ind, train_y)


# speedup vs baseline: 1.0358x; 1.0073x over previous
import functools

import jax
import jax.numpy as jnp
from jax import lax
from jax.experimental import pallas as pl
from jax.experimental.pallas import tpu as pltpu
from jax.experimental.pallas import tpu_sc as plsc

_NC, _NS, _L = 1, 16, 16
_NW = _NC * _NS
_ACTIVE = 16


_BM = 2048


def _argmin_body(x_ref, t_ref, ind_ref):
    t = t_ref[...]
    xt = x_ref[...].T
    dots = jnp.dot(t, xt, preferred_element_type=jnp.float32,
                   precision=lax.Precision.HIGHEST)
    tnorm = 0.5 * jnp.sum(t * t, axis=1)
    s = tnorm[:, None] - dots
    bmin = jnp.min(s, axis=0)
    iota = lax.broadcasted_iota(jnp.int32, s.shape, 0)
    masked = jnp.where(s == bmin[None, :], iota, jnp.int32(s.shape[0]))
    ind_ref[...] = jnp.min(masked, axis=0)


def _nearest_index(x, train_x):
    q = x.shape[0]
    m, dim = train_x.shape
    return pl.pallas_call(
        _argmin_body,
        out_shape=jax.ShapeDtypeStruct((q,), jnp.int32),
    )(x, train_x)


def _label_gather(ind, train_y):
    b = ind.shape[0]
    b_per_w = b // _ACTIVE
    mesh = plsc.VectorSubcoreMesh(core_axis_name="c", subcore_axis_name="s", num_cores=1)

    @functools.partial(
        pl.kernel,
        mesh=mesh,
        out_type=jax.ShapeDtypeStruct((b,), jnp.float32),
        scratch_types=[
            pltpu.VMEM((b_per_w,), jnp.int32),
            pltpu.VMEM((b_per_w,), jnp.float32),
            pltpu.SemaphoreType.DMA,
        ],
    )
    def gather_kernel(ind_hbm, ty_hbm, out_hbm, idx_v, out_v, sem):
        wid = lax.axis_index("s") * _NC + lax.axis_index("c")
        base = wid * b_per_w

        @pl.when(wid < _ACTIVE)
        def _():
            pltpu.sync_copy(ind_hbm.at[pl.ds(base, b_per_w)], idx_v)
            pltpu.async_copy(ty_hbm.at[idx_v], out_v, sem).wait()
            pltpu.sync_copy(out_v, out_hbm.at[pl.ds(base, b_per_w)])

    return gather_kernel(ind, train_y)


def kernel(x, train_x, train_y):
    ind = _nearest_index(x, train_x)
    return _label_gather(ind, train_y)
